# baseline (speedup 1.0000x reference)
import jax
import jax.numpy as jnp
from jax import lax
from jax.experimental import pallas as pl
from jax.experimental.pallas import tpu as pltpu
from jax.experimental.pallas import tpu_sc as plsc

_B = 16
_D = 64
_HW = 1024
_N_TOK = _B * _HW
_NC = 2
_NS = 16
_NW = _NC * _NS
_ROWS_PER_W = _B * _D // _NW
_CH_PER_SUB = _D // _NS
_ENC_PER_W = _N_TOK // _NW


def _sc_body(emb_hbm, q_hbm, enc_hbm, emb_v, stage_v, ones_v, table_sh):
    ci = lax.axis_index("c")
    sid = lax.axis_index("s")
    wid = sid * _NC + ci

    pltpu.sync_copy(
        emb_hbm.at[pl.ds(sid * _CH_PER_SUB, _CH_PER_SUB), :], emb_v
    )

    for k in range(_CH_PER_SUB):
        val = emb_v[k, :]
        for j in range(_HW // 16):
            stage_v[k, pl.ds(j * 16, 16)] = val
    pltpu.sync_copy(
        stage_v, table_sh.at[pl.ds(sid * _CH_PER_SUB, _CH_PER_SUB), :]
    )

    one = jnp.full((16,), 1.0, jnp.float32)
    for j in range(_ENC_PER_W // 16):
        ones_v[pl.ds(j * 16, 16)] = one

    plsc.subcore_barrier()

    base = wid * _ROWS_PER_W
    c0 = base % _D
    pltpu.sync_copy(
        table_sh.at[pl.ds(c0, _ROWS_PER_W), :],
        q_hbm.at[pl.ds(base, _ROWS_PER_W), :],
    )
    pltpu.sync_copy(ones_v, enc_hbm.at[pl.ds(wid * _ENC_PER_W, _ENC_PER_W)])


def kernel(x, embeddings):
    del x
    emb_splat = jnp.broadcast_to(embeddings.reshape(_D, 1), (_D, 16))
    mesh = plsc.VectorSubcoreMesh(core_axis_name="c", subcore_axis_name="s")
    q2, enc = pl.kernel(
        _sc_body,
        out_type=[
            jax.ShapeDtypeStruct((_B * _D, _HW), jnp.float32),
            jax.ShapeDtypeStruct((_N_TOK,), jnp.float32),
        ],
        mesh=mesh,
        scratch_types=[
            pltpu.VMEM((_CH_PER_SUB, 16), jnp.float32),
            pltpu.VMEM((_CH_PER_SUB, _HW), jnp.float32),
            pltpu.VMEM((_ENC_PER_W,), jnp.float32),
            pltpu.VMEM_SHARED((_D, _HW), jnp.float32),
        ],
    )(emb_splat)
    return (enc.reshape(_N_TOK, 1), q2.reshape(_B, _D, 32, 32))


# speedup vs baseline: 3.6090x; 3.6090x over previous
import jax
import jax.numpy as jnp
from jax import lax
from jax.experimental import pallas as pl
from jax.experimental.pallas import tpu as pltpu

_B = 16
_D = 64
_HW = 1024
_N_TOK = _B * _HW


def _fill_body(emb_ref, q_hbm, enc_hbm, q_v, enc_v, sem_q, sem_e):
    col = emb_ref[...]
    q_v[...] = lax.broadcast_in_dim(col, (_B, _D, _HW), (1, 2))
    enc_v[...] = jnp.full((128, 128), 1.0, jnp.float32)
    cq = pltpu.make_async_copy(q_v, q_hbm, sem_q)
    ce = pltpu.make_async_copy(enc_v, enc_hbm, sem_e)
    cq.start()
    ce.start()
    cq.wait()
    ce.wait()


def kernel(x, embeddings):
    del x
    emb_col = embeddings.reshape(_D, 1)
    q3, enc2 = pl.pallas_call(
        _fill_body,
        in_specs=[pl.BlockSpec(memory_space=pltpu.VMEM)],
        out_specs=[
            pl.BlockSpec(memory_space=pl.ANY),
            pl.BlockSpec(memory_space=pl.ANY),
        ],
        out_shape=[
            jax.ShapeDtypeStruct((_B, _D, _HW), jnp.float32),
            jax.ShapeDtypeStruct((128, 128), jnp.float32),
        ],
        scratch_shapes=[
            pltpu.VMEM((_B, _D, _HW), jnp.float32),
            pltpu.VMEM((128, 128), jnp.float32),
            pltpu.SemaphoreType.DMA,
            pltpu.SemaphoreType.DMA,
        ],
    )(emb_col)
    quantized = q3.reshape(_B, _D, 32, 32)
    encodings = enc2.reshape(_N_TOK, 1)
    return (encodings, quantized)
